# baseline (speedup 1.0000x reference)
import jax
import jax.numpy as jnp
from jax.experimental import pallas as pl
from jax.experimental.pallas import tpu as pltpu

B, L, N = 4, 2048, 32
D, P = 768, 720
E = 16
CID, HID = 64, 128
BN = B * N
NBUF = 3


def _bdot(a, b):
    return jnp.dot(a.astype(jnp.bfloat16), b.astype(jnp.bfloat16),
                   preferred_element_type=jnp.float32)


def _stats(xt):
    mean = jnp.mean(xt, axis=1, keepdims=True)
    xm = xt - mean
    var = jnp.sum(xm * xm, axis=1, keepdims=True) / (L - 1)
    std = jnp.sqrt(var) + 1e-6
    return mean, std


def _fused_kernel(xt_ref, ci_ref, rw1_ref, rb1_ref, rw2_ref, rb2_ref,
                  bexp_ref, t1b_ref, t2b_ref, pb_ref,
                  wexp_hbm, t1w_hbm, t2w_hbm, pw_hbm,
                  out_ref,
                  wbuf, t1v, t2v, pwv, xn_ref, acc_ref, g_ref,
                  wsem, hsem):
    def wcopy(e, slot):
        return pltpu.make_async_copy(wexp_hbm.at[e], wbuf.at[slot],
                                     wsem.at[slot])

    wcopy(0, 0).start()
    wcopy(1, 1).start()

    mean, std = _stats(xt_ref[...])
    xn_ref[...] = ((xt_ref[...] - mean) / std).astype(jnp.bfloat16)
    h = jnp.maximum(
        jnp.dot(ci_ref[...], rw1_ref[...],
                preferred_element_type=jnp.float32) + rb1_ref[...], 0.0)
    logits = jnp.dot(h, rw2_ref[...],
                     preferred_element_type=jnp.float32) + rb2_ref[...]
    m = jnp.max(logits, axis=1, keepdims=True)
    ex = jnp.exp(logits - m)
    g = ex / jnp.sum(ex, axis=1, keepdims=True)
    g_ref[...] = jnp.concatenate([g] * B, axis=0)
    acc_ref[...] = jnp.zeros_like(acc_ref)

    for e in range(E):
        slot = e % NBUF
        if e + 2 < E:
            wcopy(e + 2, (e + 2) % NBUF).start()
        if e == E - 2:
            pltpu.make_async_copy(t1w_hbm, t1v, hsem.at[0]).start()
            pltpu.make_async_copy(t2w_hbm, t2v, hsem.at[1]).start()
            pltpu.make_async_copy(pw_hbm, pwv, hsem.at[2]).start()
        wcopy(e, slot).wait()
        scale = g_ref[:, e:e + 1]
        z = jnp.dot(xn_ref[...], wbuf[slot].astype(jnp.bfloat16),
                    preferred_element_type=jnp.float32)
        acc_ref[...] += z * scale

    pltpu.make_async_copy(t1w_hbm, t1v, hsem.at[0]).wait()
    pltpu.make_async_copy(t2w_hbm, t2v, hsem.at[1]).wait()
    pltpu.make_async_copy(pw_hbm, pwv, hsem.at[2]).wait()

    emb = acc_ref[...] + jnp.concatenate([bexp_ref[...]] * B, axis=0)
    t = jnp.maximum(_bdot(emb, t1v[...]) + t1b_ref[...], 0.0)
    x2 = _bdot(t, t2v[...]) + t2b_ref[...] + emb
    pred = _bdot(x2, pwv[...]) + pb_ref[...]
    out_ref[...] = pred * std + mean


@jax.jit
def kernel(x, CI, rW1, rb1, rW2, rb2, Wexp, Bexp, T1w, T1b, T2w, T2b, Pw, Pb):
    xt = jnp.transpose(x, (0, 2, 1)).reshape(BN, L)

    vmem = pl.BlockSpec(memory_space=pltpu.MemorySpace.VMEM)
    hbm = pl.BlockSpec(memory_space=pltpu.MemorySpace.HBM)
    ins = (xt, CI, rW1, rb1.reshape(1, HID), rW2, rb2.reshape(1, E),
           Bexp, T1b.reshape(1, D), T2b.reshape(1, D), Pb.reshape(1, P),
           Wexp, T1w, T2w, Pw)
    specs = [vmem] * 10 + [hbm] * 4

    out = pl.pallas_call(
        _fused_kernel,
        in_specs=specs,
        out_specs=pl.BlockSpec(memory_space=pltpu.MemorySpace.VMEM),
        out_shape=jax.ShapeDtypeStruct((BN, P), jnp.float32),
        scratch_shapes=[
            pltpu.VMEM((NBUF, L, D), jnp.float32),
            pltpu.VMEM((D, D), jnp.float32),
            pltpu.VMEM((D, D), jnp.float32),
            pltpu.VMEM((D, P), jnp.float32),
            pltpu.VMEM((BN, L), jnp.bfloat16),
            pltpu.VMEM((BN, D), jnp.float32),
            pltpu.VMEM((BN, E), jnp.float32),
            pltpu.SemaphoreType.DMA((NBUF,)),
            pltpu.SemaphoreType.DMA((3,)),
        ],
    )(*ins)

    return jnp.transpose(out.reshape(B, N, P), (0, 2, 1))


# speedup vs baseline: 1.0275x; 1.0275x over previous
import jax
import jax.numpy as jnp
from jax.experimental import pallas as pl
from jax.experimental.pallas import tpu as pltpu

B, L, N = 4, 2048, 32
D, P = 768, 720
E = 16
CID, HID = 64, 128
BN = B * N


def _bdot(a, b):
    return jnp.dot(a.astype(jnp.bfloat16), b.astype(jnp.bfloat16),
                   preferred_element_type=jnp.float32)


def _stats(xt):
    mean = jnp.mean(xt, axis=1, keepdims=True)
    xm = xt - mean
    var = jnp.sum(xm * xm, axis=1, keepdims=True) / (L - 1)
    std = jnp.sqrt(var) + 1e-6
    return mean, std


def _fused_kernel(xt_ref, ci_ref, rw1_ref, rb1_ref, rw2_ref, rb2_ref,
                  wexp_ref, bexp_ref, t1w_ref, t1b_ref, t2w_ref, t2b_ref,
                  pw_ref, pb_ref, out_ref, xn_ref, acc_ref, g_ref):
    e = pl.program_id(0)

    @pl.when(e == 0)
    def _init():
        mean, std = _stats(xt_ref[...])
        xn_ref[...] = (xt_ref[...] - mean) / std
        h = jnp.maximum(
            jnp.dot(ci_ref[...], rw1_ref[...],
                    preferred_element_type=jnp.float32) + rb1_ref[...], 0.0)
        logits = jnp.dot(h, rw2_ref[...],
                         preferred_element_type=jnp.float32) + rb2_ref[...]
        m = jnp.max(logits, axis=1, keepdims=True)
        ex = jnp.exp(logits - m)
        g = ex / jnp.sum(ex, axis=1, keepdims=True)
        g_ref[...] = jnp.concatenate([g] * B, axis=0)
        acc_ref[...] = jnp.zeros_like(acc_ref)

    lane = jax.lax.broadcasted_iota(jnp.int32, (1, E), 1)
    scale = jnp.sum(jnp.where(lane == e, g_ref[...], 0.0), axis=1,
                    keepdims=True)
    z = jnp.dot(xn_ref[...], wexp_ref[0],
                preferred_element_type=jnp.float32)
    acc_ref[...] += z * scale

    @pl.when(e == E - 1)
    def _head():
        emb = acc_ref[...] + jnp.concatenate([bexp_ref[...]] * B, axis=0)
        t = jnp.maximum(_bdot(emb, t1w_ref[...]) + t1b_ref[...], 0.0)
        x2 = _bdot(t, t2w_ref[...]) + t2b_ref[...] + emb
        pred = _bdot(x2, pw_ref[...]) + pb_ref[...]
        mean, std = _stats(xt_ref[...])
        out_ref[...] = pred * std + mean


@jax.jit
def kernel(x, CI, rW1, rb1, rW2, rb2, Wexp, Bexp, T1w, T1b, T2w, T2b, Pw, Pb):
    xt = jnp.transpose(x, (0, 2, 1)).reshape(BN, L)

    const = lambda arr: pl.BlockSpec(arr.shape, lambda e: (0,) * arr.ndim)
    ins = (xt, CI, rW1, rb1.reshape(1, HID), rW2, rb2.reshape(1, E),
           Wexp, Bexp, T1w, T1b.reshape(1, D), T2w, T2b.reshape(1, D),
           Pw, Pb.reshape(1, P))
    specs = [const(a) for a in ins]
    specs[6] = pl.BlockSpec((1, L, D), lambda e: (e, 0, 0))

    out = pl.pallas_call(
        _fused_kernel,
        grid=(E,),
        in_specs=specs,
        out_specs=pl.BlockSpec((BN, P), lambda e: (0, 0)),
        out_shape=jax.ShapeDtypeStruct((BN, P), jnp.float32),
        scratch_shapes=[
            pltpu.VMEM((BN, L), jnp.float32),
            pltpu.VMEM((BN, D), jnp.float32),
            pltpu.VMEM((BN, E), jnp.float32),
        ],
        compiler_params=pltpu.CompilerParams(
            dimension_semantics=("arbitrary",),
        ),
    )(*ins)

    return jnp.transpose(out.reshape(B, N, P), (0, 2, 1))
